# baseline (speedup 1.0000x reference)
import functools

import jax
import jax.numpy as jnp
from jax.experimental import pallas as pl
from jax.experimental.pallas import tpu as pltpu


def _masked_linear_kern(x_ref, w_ref, m_ref, b_ref, o_ref, wm_ref):
    @pl.when(pl.program_id(0) == 0)
    def _():
        wm_ref[...] = (w_ref[...] * m_ref[...]).astype(jnp.bfloat16)

    xb = x_ref[...].astype(jnp.bfloat16)
    acc = jax.lax.dot_general(
        xb, wm_ref[...],
        dimension_numbers=(((1,), (1,)), ((), ())),
        preferred_element_type=jnp.float32,
    )
    o_ref[...] = acc + b_ref[...]


@functools.partial(jax.jit, static_argnames=("bm",))
def _masked_linear(x2, W, b2, binary_mask, bm=512):
    M, K = x2.shape
    N = W.shape[0]
    return pl.pallas_call(
        _masked_linear_kern,
        grid=(M // bm,),
        in_specs=[
            pl.BlockSpec((bm, K), lambda i: (i, 0)),
            pl.BlockSpec((N, K), lambda i: (0, 0)),
            pl.BlockSpec((N, K), lambda i: (0, 0)),
            pl.BlockSpec((1, N), lambda i: (0, 0)),
        ],
        out_specs=pl.BlockSpec((bm, N), lambda i: (i, 0)),
        out_shape=jax.ShapeDtypeStruct((M, N), jnp.float32),
        scratch_shapes=[pltpu.VMEM((N, K), jnp.bfloat16)],
    )(x2, W, binary_mask, b2)


def kernel(x, W, b, binary_mask):
    B, S, D = x.shape
    N = W.shape[0]
    out = _masked_linear(x.reshape(B * S, D), W, b.reshape(1, N), binary_mask)
    return out.reshape(B, S, N)


# speedup vs baseline: 1.2666x; 1.2666x over previous
import jax
import jax.numpy as jnp
from jax.experimental import pallas as pl
from jax.experimental.pallas import tpu as pltpu

BM = 1024
NBUF = 3


def _mp_kern(x_hbm, w_hbm, m_hbm, b_hbm, o_hbm,
             wvm, mvm, bvm, wm, xbuf, obuf,
             wsems, in_sems, out_sems):
    M = x_hbm.shape[0]
    T = M // BM

    w_cp = pltpu.make_async_copy(w_hbm, wvm, wsems.at[0])
    m_cp = pltpu.make_async_copy(m_hbm, mvm, wsems.at[1])
    b_cp = pltpu.make_async_copy(b_hbm, bvm, wsems.at[2])
    w_cp.start()
    m_cp.start()
    b_cp.start()
    x_cps = []
    for t in range(min(NBUF, T)):
        cp = pltpu.make_async_copy(
            x_hbm.at[pl.ds(t * BM, BM), :], xbuf.at[t % NBUF], in_sems.at[t % NBUF])
        cp.start()
        x_cps.append(cp)

    w_cp.wait()
    m_cp.wait()
    wm[...] = (wvm[...] * mvm[...]).astype(jnp.bfloat16)
    b_cp.wait()

    out_cps = [None] * NBUF
    for t in range(T):
        buf = t % NBUF
        x_cps[t].wait()
        if out_cps[buf] is not None:
            out_cps[buf].wait()
        xb = xbuf[buf].astype(jnp.bfloat16)
        acc = jax.lax.dot_general(
            xb, wm[...],
            dimension_numbers=(((1,), (1,)), ((), ())),
            preferred_element_type=jnp.float32,
        )
        obuf[buf] = acc + bvm[...]
        ocp = pltpu.make_async_copy(
            obuf.at[buf], o_hbm.at[pl.ds(t * BM, BM), :], out_sems.at[buf])
        ocp.start()
        out_cps[buf] = ocp
        nxt = t + NBUF
        if nxt < T:
            cp = pltpu.make_async_copy(
                x_hbm.at[pl.ds(nxt * BM, BM), :], xbuf.at[buf], in_sems.at[buf])
            cp.start()
            x_cps.append(cp)

    for buf in range(min(NBUF, T)):
        if out_cps[buf] is not None:
            out_cps[buf].wait()


def _masked_linear(x2, W, b2, binary_mask):
    M, K = x2.shape
    N = W.shape[0]
    return pl.pallas_call(
        _mp_kern,
        in_specs=[
            pl.BlockSpec(memory_space=pl.ANY),
            pl.BlockSpec(memory_space=pl.ANY),
            pl.BlockSpec(memory_space=pl.ANY),
            pl.BlockSpec(memory_space=pl.ANY),
        ],
        out_specs=pl.BlockSpec(memory_space=pl.ANY),
        out_shape=jax.ShapeDtypeStruct((M, N), jnp.float32),
        scratch_shapes=[
            pltpu.VMEM((N, K), jnp.float32),
            pltpu.VMEM((N, K), jnp.float32),
            pltpu.VMEM((1, N), jnp.float32),
            pltpu.VMEM((N, K), jnp.bfloat16),
            pltpu.VMEM((NBUF, BM, K), jnp.float32),
            pltpu.VMEM((NBUF, BM, N), jnp.float32),
            pltpu.SemaphoreType.DMA((3,)),
            pltpu.SemaphoreType.DMA((NBUF,)),
            pltpu.SemaphoreType.DMA((NBUF,)),
        ],
    )(x2, W, binary_mask, b2)


def kernel(x, W, b, binary_mask):
    B, S, D = x.shape
    N = W.shape[0]
    out = _masked_linear(x.reshape(B * S, D), W, b.reshape(1, N), binary_mask)
    return out.reshape(B, S, N)
